# baseline (speedup 1.0000x reference)
import functools

import jax
import jax.numpy as jnp
from jax import lax
from jax.experimental import pallas as pl
from jax.experimental.pallas import tpu as pltpu
from jax.experimental.pallas import tpu_sc as plsc

NUM_CORES = 2
NUM_SUBCORES = 16
NUM_WORKERS = NUM_CORES * NUM_SUBCORES
LANES = 16
IDX_SPLIT = 2


def _make_sc_kernel(B, S, V, E):
    n = B * S
    assert B % NUM_WORKERS == 0
    assert S % IDX_SPLIT == 0
    seqs_per_worker = B // NUM_WORKERS
    W = S // IDX_SPLIT

    mesh = plsc.VectorSubcoreMesh(core_axis_name="c", subcore_axis_name="s")

    @functools.partial(
        pl.kernel,
        mesh=mesh,
        out_type=jax.ShapeDtypeStruct((n, E), jnp.float32),
        compiler_params=pltpu.CompilerParams(use_tc_tiling_on_sc=False),
        scratch_types=[
            pltpu.VMEM((IDX_SPLIT, W), jnp.int32),
            pltpu.VMEM((S, E), jnp.float32),
            pltpu.VMEM((S, E), jnp.float32),
        ],
    )
    def k(ids_hbm, tok_hbm, pos_hbm, out_hbm, idx_v, rows_v, pos_v):
        wid = lax.axis_index("s") * NUM_CORES + lax.axis_index("c")
        pltpu.sync_copy(pos_hbm.at[pl.ds(0, S)], pos_v)

        @pl.loop(0, seqs_per_worker)
        def _(w):
            seq = wid * seqs_per_worker + w
            base = seq * S
            pltpu.sync_copy(ids_hbm.at[seq], idx_v)
            for j in range(IDX_SPLIT):
                pltpu.sync_copy(
                    tok_hbm.at[idx_v.at[j]], rows_v.at[pl.ds(j * W, W)]
                )

            @pl.loop(0, S)
            def _(r):
                for c in range(E // LANES):
                    sl = pl.ds(c * LANES, LANES)
                    rows_v[r, sl] = rows_v[r, sl] + pos_v[r, sl]

            pltpu.sync_copy(rows_v, out_hbm.at[pl.ds(base, S)])

    return k


def kernel(input_ids, token_table, position_table):
    B, S = input_ids.shape
    V, E = token_table.shape
    ids = input_ids.astype(jnp.int32).reshape(B, IDX_SPLIT, S // IDX_SPLIT)
    sc = _make_sc_kernel(B, S, V, E)
    out = sc(ids, token_table, position_table)
    return out.reshape(B, S, E)


# speedup vs baseline: 1.2438x; 1.2438x over previous
import functools

import jax
import jax.numpy as jnp
from jax import lax
from jax.experimental import pallas as pl
from jax.experimental.pallas import tpu as pltpu
from jax.experimental.pallas import tpu_sc as plsc

NUM_CORES = 2
NUM_SUBCORES = 16
NUM_WORKERS = NUM_CORES * NUM_SUBCORES
CHUNK = 128


def _make_sc_kernel(B, S, V, E):
    n = B * S
    num_chunks = n // CHUNK
    assert n % CHUNK == 0 and num_chunks % NUM_WORKERS == 0
    assert S % 8 == 0
    cpw = num_chunks // NUM_WORKERS
    assert cpw % 2 == 0

    mesh = plsc.VectorSubcoreMesh(core_axis_name="c", subcore_axis_name="s")

    @functools.partial(
        pl.kernel,
        mesh=mesh,
        out_type=jax.ShapeDtypeStruct((n, E), jnp.float32),
        compiler_params=pltpu.CompilerParams(use_tc_tiling_on_sc=False),
        scratch_types=[
            pltpu.VMEM((cpw, CHUNK), jnp.int32),
            pltpu.VMEM((2 * S, E), jnp.float32),
            pltpu.VMEM((CHUNK,), jnp.int32),
            pltpu.VMEM((CHUNK, E), jnp.float32),
            pltpu.VMEM((CHUNK, E), jnp.float32),
            pltpu.VMEM_SHARED((NUM_SUBCORES, 2, CHUNK, E), jnp.float32),
            pltpu.SemaphoreType.DMA,
            pltpu.SemaphoreType.DMA,
            pltpu.SemaphoreType.DMA,
            pltpu.SemaphoreType.DMA,
        ],
    )
    def k(ids_hbm, tok_hbm, pos_hbm, iota_hbm, out_hbm,
          idx_v, pos2_v, iota_v, g0, g1, stage,
          gsem0, gsem1, osem0, osem1):
        sid = lax.axis_index("s")
        wid = sid * NUM_CORES + lax.axis_index("c")
        gbuf = (g0, g1)
        sbuf = (stage.at[sid, 0], stage.at[sid, 1])
        gsem = (gsem0, gsem1)
        osem = (osem0, osem1)

        pltpu.sync_copy(ids_hbm.at[pl.ds(wid * cpw, cpw)], idx_v)
        pltpu.sync_copy(pos_hbm.at[pl.ds(0, S)], pos2_v.at[pl.ds(0, S)])
        pltpu.sync_copy(pos_hbm.at[pl.ds(0, S)], pos2_v.at[pl.ds(S, S)])
        pltpu.sync_copy(iota_hbm, iota_v)

        def gather_copy(t, b):
            return pltpu.make_async_copy(
                tok_hbm.at[idx_v.at[t]], gbuf[b], gsem[b])

        def out_copy(t, b):
            base = (wid * cpw + t) * CHUNK
            return pltpu.make_async_copy(
                sbuf[b], out_hbm.at[pl.ds(base, CHUNK)], osem[b])

        for b in range(2):
            gather_copy(b, b).start()

        @pl.loop(0, cpw, step=2)
        def _(i):
            for b in range(2):
                t = i + b
                g = wid * cpw + t
                phase = lax.rem(g * CHUNK, S)

                @pl.when(t >= 2)
                def _():
                    out_copy(t - 2, b).wait()

                pltpu.sync_copy(pos2_v.at[pl.ds(phase, CHUNK)], sbuf[b])
                gather_copy(t, b).wait()
                pltpu.sync_copy(gbuf[b], sbuf[b].at[iota_v], add=True)
                out_copy(t, b).start()

                @pl.when(t + 2 < cpw)
                def _():
                    gather_copy(t + 2, b).start()

        for b in range(2):
            out_copy(cpw - 2 + b, b).wait()

    return k


def kernel(input_ids, token_table, position_table):
    B, S = input_ids.shape
    V, E = token_table.shape
    n = B * S
    ids = input_ids.astype(jnp.int32).reshape(n // CHUNK, CHUNK)
    iota = jnp.arange(CHUNK, dtype=jnp.int32)
    sc = _make_sc_kernel(B, S, V, E)
    out = sc(ids, token_table, position_table, iota)
    return out.reshape(B, S, E)
